# baseline (speedup 1.0000x reference)
import functools

import jax
import jax.numpy as jnp
from jax import lax
from jax.experimental import pallas as pl
from jax.experimental.pallas import tpu as pltpu
from jax.experimental.pallas import tpu_sc as plsc


def _sc_dims():
    try:
        info = plsc.get_sparse_core_info()
        return info.num_cores, info.num_subcores
    except Exception:
        return 2, 16


def _sc_gather_rows(tab, ids, *, chunk, nbuf=3):
    n_tok, h = ids.shape[0], tab.shape[1]
    nc, ns = _sc_dims()
    nw = nc * ns
    assert n_tok % (nw * chunk) == 0
    per_w = n_tok // nw
    n_chunks = per_w // chunk
    mesh = plsc.VectorSubcoreMesh(core_axis_name="c", subcore_axis_name="s")

    @functools.partial(
        pl.kernel,
        mesh=mesh,
        out_type=jax.ShapeDtypeStruct((n_tok, h), tab.dtype),
        scratch_types=(
            [pltpu.VMEM((per_w,), jnp.int32)]
            + [pltpu.VMEM((chunk, h), tab.dtype) for _ in range(nbuf)]
            + [pltpu.SemaphoreType.DMA for _ in range(2 * nbuf)]
        ),
    )
    def k(tab_hbm, ids_hbm, out_hbm, idx_v, *rest):
        bufs, sems = rest[:nbuf], rest[nbuf:]
        gsem, wsem = sems[:nbuf], sems[nbuf:]
        wid = lax.axis_index("s") * nc + lax.axis_index("c")
        base = wid * per_w
        pltpu.sync_copy(ids_hbm.at[pl.ds(base, per_w)], idx_v)

        def gstart(c):
            return pltpu.async_copy(
                tab_hbm.at[idx_v.at[pl.ds(c * chunk, chunk)]],
                bufs[c % nbuf], gsem[c % nbuf])

        g = [None] * n_chunks
        w = [None] * n_chunks
        for c in range(min(nbuf - 1, n_chunks)):
            g[c] = gstart(c)
        for c in range(n_chunks):
            g[c].wait()
            w[c] = pltpu.async_copy(
                bufs[c % nbuf], out_hbm.at[pl.ds(base + c * chunk, chunk)],
                wsem[c % nbuf])
            nxt = c + nbuf - 1
            if nxt < n_chunks:
                if nxt >= nbuf:
                    w[nxt - nbuf].wait()
                g[nxt] = gstart(nxt)
        for c in range(max(0, n_chunks - nbuf), n_chunks):
            w[c].wait()

    return k(tab, ids)


def _tc_body(t_blk, h, w_ref, mt_ref, bert_ref, tid_ref, lid_ref, emb_ref,
             *rest):
    out_ref = rest[-1]
    bert_bf = bert_ref[0].astype(jnp.bfloat16)
    acc = jnp.dot(w_ref[...], bert_bf, preferred_element_type=jnp.float32)
    iota = lax.broadcasted_iota(jnp.int32, (128, t_blk), 0)
    oh = ((iota == tid_ref[0]) | (iota == lid_ref[0] + 16))
    acc = acc + jnp.dot(mt_ref[...], oh.astype(jnp.bfloat16),
                        preferred_element_type=jnp.float32)
    raw = emb_ref[...]
    lo = lax.bitcast_convert_type(raw << 16, jnp.float32)
    hi = lax.bitcast_convert_type(raw & jnp.int32(-65536), jnp.float32)
    out_ref[0, : h // 2, :] = acc[: h // 2, :] + lo.T
    out_ref[0, h // 2 :, :] = acc[h // 2 :, :] + hi.T


def _tc_proj_add(w_bf, mt_bf, bert, tids3, lids3, emb, *, t_blk,
                 b_off, nb, prev=None):
    b, d, t = bert.shape
    h = w_bf.shape[0]
    nt = t // t_blk
    grid = (nb, nt)
    in_specs = [
        pl.BlockSpec((h, d), lambda i, j: (0, 0)),
        pl.BlockSpec((h, 128), lambda i, j: (0, 0)),
        pl.BlockSpec((1, d, t_blk), lambda i, j: (b_off + i, 0, j)),
        pl.BlockSpec((1, 1, t_blk), lambda i, j: (b_off + i, 0, j)),
        pl.BlockSpec((1, 1, t_blk), lambda i, j: (b_off + i, 0, j)),
        pl.BlockSpec((t_blk, h // 2), lambda i, j: (i * nt + j, 0)),
    ]
    args = [w_bf, mt_bf, bert, tids3, lids3, emb]
    aliases = {}
    if prev is not None:
        in_specs.append(pl.BlockSpec(memory_space=pl.ANY))
        args.append(prev)
        aliases = {6: 0}
    return pl.pallas_call(
        functools.partial(_tc_body, t_blk, h),
        grid=grid,
        in_specs=in_specs,
        out_specs=pl.BlockSpec((1, h, t_blk), lambda i, j: (b_off + i, 0, j)),
        out_shape=jax.ShapeDtypeStruct((b, h, t), jnp.float32),
        input_output_aliases=aliases,
        compiler_params=pltpu.CompilerParams(
            dimension_semantics=("parallel", "parallel")),
    )(*args)


def kernel(phoneme_ids, tone_ids, language_ids, bert_feats,
           phoneme_table, tone_table, language_table, W_bert):
    b, t = phoneme_ids.shape
    h = phoneme_table.shape[1]
    n = b * t
    pids = phoneme_ids.reshape(n).astype(jnp.int32)
    tids3 = tone_ids.reshape(b, 1, t).astype(jnp.int32)
    lids3 = language_ids.reshape(b, 1, t).astype(jnp.int32)

    n_tone, n_lang = tone_table.shape[0], language_table.shape[0]
    mt = jnp.zeros((128, h), jnp.float32)
    mt = mt.at[:n_tone].set(tone_table).at[16:16 + n_lang].set(language_table)
    mt_bf = mt.T.astype(jnp.bfloat16)
    w_bf = W_bert.astype(jnp.bfloat16)

    pb = phoneme_table.astype(jnp.bfloat16)
    lo16 = lax.bitcast_convert_type(pb[:, : h // 2], jnp.uint16)
    hi16 = lax.bitcast_convert_type(pb[:, h // 2 :], jnp.uint16)
    ptab_pack = lo16.astype(jnp.int32) | (hi16.astype(jnp.int32) << 16)

    n_slices = 4
    bs = b // n_slices
    ns = bs * t
    out = None
    for s in range(n_slices):
        pids_s = lax.slice(pids, (s * ns,), ((s + 1) * ns,))
        emb_s = _sc_gather_rows(ptab_pack, pids_s, chunk=64)
        out = _tc_proj_add(w_bf, mt_bf, bert_feats, tids3, lids3, emb_s,
                           t_blk=1024, b_off=s * bs, nb=bs, prev=out)
    return out


# speedup vs baseline: 1.0391x; 1.0391x over previous
import functools

import jax
import jax.numpy as jnp
from jax import lax
from jax.experimental import pallas as pl
from jax.experimental.pallas import tpu as pltpu
from jax.experimental.pallas import tpu_sc as plsc


def _sc_dims():
    try:
        info = plsc.get_sparse_core_info()
        return info.num_cores, info.num_subcores
    except Exception:
        return 2, 16


def _sc_gather_rows(tab, ids, *, chunk, nbuf=3):
    n_tok, h = ids.shape[0], tab.shape[1]
    nc, ns = _sc_dims()
    nw = nc * ns
    assert n_tok % (nw * chunk) == 0
    per_w = n_tok // nw
    n_chunks = per_w // chunk
    mesh = plsc.VectorSubcoreMesh(core_axis_name="c", subcore_axis_name="s")

    @functools.partial(
        pl.kernel,
        mesh=mesh,
        out_type=jax.ShapeDtypeStruct((n_tok, h), tab.dtype),
        scratch_types=(
            [pltpu.VMEM((per_w,), jnp.int32)]
            + [pltpu.VMEM((chunk, h), tab.dtype) for _ in range(nbuf)]
            + [pltpu.SemaphoreType.DMA for _ in range(2 * nbuf)]
        ),
    )
    def k(tab_hbm, ids_hbm, out_hbm, idx_v, *rest):
        bufs, sems = rest[:nbuf], rest[nbuf:]
        gsem, wsem = sems[:nbuf], sems[nbuf:]
        wid = lax.axis_index("s") * nc + lax.axis_index("c")
        base = wid * per_w
        pltpu.sync_copy(ids_hbm.at[pl.ds(base, per_w)], idx_v)

        def gstart(c):
            return pltpu.async_copy(
                tab_hbm.at[idx_v.at[pl.ds(c * chunk, chunk)]],
                bufs[c % nbuf], gsem[c % nbuf])

        g = [None] * n_chunks
        w = [None] * n_chunks
        for c in range(min(nbuf - 1, n_chunks)):
            g[c] = gstart(c)
        for c in range(n_chunks):
            g[c].wait()
            w[c] = pltpu.async_copy(
                bufs[c % nbuf], out_hbm.at[pl.ds(base + c * chunk, chunk)],
                wsem[c % nbuf])
            nxt = c + nbuf - 1
            if nxt < n_chunks:
                if nxt >= nbuf:
                    w[nxt - nbuf].wait()
                g[nxt] = gstart(nxt)
        for c in range(max(0, n_chunks - nbuf), n_chunks):
            w[c].wait()

    return k(tab, ids)


def _tc_body(t_blk, h, w_ref, mt_ref, bert_ref, tid_ref, lid_ref, emb_ref,
             *rest):
    out_ref = rest[-1]
    bert_bf = bert_ref[0].astype(jnp.bfloat16)
    acc = jnp.dot(w_ref[...], bert_bf, preferred_element_type=jnp.float32)
    iota = lax.broadcasted_iota(jnp.int32, (128, t_blk), 0)
    oh = ((iota == tid_ref[0]) | (iota == lid_ref[0] + 16))
    acc = acc + jnp.dot(mt_ref[...], oh.astype(jnp.bfloat16),
                        preferred_element_type=jnp.float32)
    raw = emb_ref[...]
    lo = lax.bitcast_convert_type(raw << 16, jnp.float32)
    hi = lax.bitcast_convert_type(raw & jnp.int32(-65536), jnp.float32)
    out_ref[0, : h // 2, :] = acc[: h // 2, :] + lo.T
    out_ref[0, h // 2 :, :] = acc[h // 2 :, :] + hi.T


def _tc_proj_add(w_bf, mt_bf, bert, tids3, lids3, emb, *, t_blk,
                 b_off, nb, prev=None):
    b, d, t = bert.shape
    h = w_bf.shape[0]
    nt = t // t_blk
    grid = (nb, nt)
    in_specs = [
        pl.BlockSpec((h, d), lambda i, j: (0, 0)),
        pl.BlockSpec((h, 128), lambda i, j: (0, 0)),
        pl.BlockSpec((1, d, t_blk), lambda i, j: (b_off + i, 0, j)),
        pl.BlockSpec((1, 1, t_blk), lambda i, j: (b_off + i, 0, j)),
        pl.BlockSpec((1, 1, t_blk), lambda i, j: (b_off + i, 0, j)),
        pl.BlockSpec((t_blk, h // 2), lambda i, j: (i * nt + j, 0)),
    ]
    args = [w_bf, mt_bf, bert, tids3, lids3, emb]
    aliases = {}
    if prev is not None:
        in_specs.append(pl.BlockSpec(memory_space=pl.ANY))
        args.append(prev)
        aliases = {6: 0}
    return pl.pallas_call(
        functools.partial(_tc_body, t_blk, h),
        grid=grid,
        in_specs=in_specs,
        out_specs=pl.BlockSpec((1, h, t_blk), lambda i, j: (b_off + i, 0, j)),
        out_shape=jax.ShapeDtypeStruct((b, h, t), jnp.float32),
        input_output_aliases=aliases,
        compiler_params=pltpu.CompilerParams(
            dimension_semantics=("parallel", "parallel")),
    )(*args)


def kernel(phoneme_ids, tone_ids, language_ids, bert_feats,
           phoneme_table, tone_table, language_table, W_bert):
    b, t = phoneme_ids.shape
    h = phoneme_table.shape[1]
    n = b * t
    pids = phoneme_ids.reshape(n).astype(jnp.int32)
    tids3 = tone_ids.reshape(b, 1, t).astype(jnp.int32)
    lids3 = language_ids.reshape(b, 1, t).astype(jnp.int32)

    n_tone, n_lang = tone_table.shape[0], language_table.shape[0]
    mt = jnp.zeros((128, h), jnp.float32)
    mt = mt.at[:n_tone].set(tone_table).at[16:16 + n_lang].set(language_table)
    mt_bf = mt.T.astype(jnp.bfloat16)
    w_bf = W_bert.astype(jnp.bfloat16)

    pb = phoneme_table.astype(jnp.bfloat16)
    lo16 = lax.bitcast_convert_type(pb[:, : h // 2], jnp.uint16)
    hi16 = lax.bitcast_convert_type(pb[:, h // 2 :], jnp.uint16)
    ptab_pack = lo16.astype(jnp.int32) | (hi16.astype(jnp.int32) << 16)

    slice_sizes = (4, 12)
    out = None
    b_off = 0
    for bs in slice_sizes:
        ns = bs * t
        pids_s = lax.slice(pids, (b_off * t,), (b_off * t + ns,))
        emb_s = _sc_gather_rows(ptab_pack, pids_s, chunk=64)
        out = _tc_proj_add(w_bf, mt_bf, bert_feats, tids3, lids3, emb_s,
                           t_blk=1024, b_off=b_off, nb=bs, prev=out)
        b_off += bs
    return out
